# baseline (speedup 1.0000x reference)
import jax
import jax.numpy as jnp
from jax.experimental import pallas as pl
from jax.experimental.pallas import tpu as pltpu


_GROUP = 8


def _nconv_block_kernel(a_ref, x_ref, o_ref):
    j = pl.program_id(1)
    hB, _, L = o_ref.shape
    at = jnp.transpose(a_ref[...], (1, 0)).astype(jnp.bfloat16)
    g = _GROUP if hB % _GROUP == 0 else 1
    for i in range(0, hB, g):
        xg = jnp.concatenate(
            [x_ref[j * hB + i + k].astype(jnp.bfloat16) for k in range(g)],
            axis=1)
        y = jax.lax.dot_general(
            at, xg,
            dimension_numbers=(((1,), (0,)), ((), ())),
            preferred_element_type=jnp.float32,
        )
        for k in range(g):
            o_ref[i + k] = y[:, k * L:(k + 1) * L]


def _pick_bb(bc):
    for cand in (64, 32, 16, 8, 4, 2):
        if bc % (2 * cand) == 0:
            return cand
    return 1


def kernel(x, A):
    N, C, V, L = x.shape
    Va, W = A.shape
    assert Va == V
    Bc = N * C
    xb = x.reshape(Bc, V, L)
    bB = _pick_bb(Bc)
    grid = (Bc // bB, 2)

    itemsize = jnp.dtype(x.dtype).itemsize
    needed = 2 * bB * (V + W) * L * itemsize + 2 * V * W * itemsize

    out = pl.pallas_call(
        _nconv_block_kernel,
        out_shape=jax.ShapeDtypeStruct((Bc, W, L), x.dtype),
        grid=grid,
        in_specs=[
            pl.BlockSpec((V, W), lambda i, j: (0, 0)),
            pl.BlockSpec((bB, V, L), lambda i, j: (i, 0, 0)),
        ],
        out_specs=pl.BlockSpec((bB // 2, W, L),
                               lambda i, j: (2 * i + j, 0, 0)),
        compiler_params=pltpu.CompilerParams(
            dimension_semantics=("arbitrary", "arbitrary"),
            vmem_limit_bytes=int(needed + (6 << 20)),
        ),
    )(A, xb)
    return out.reshape(N, C, W, L)


# speedup vs baseline: 1.3962x; 1.3962x over previous
import jax
import jax.numpy as jnp
from jax.experimental import pallas as pl
from jax.experimental.pallas import tpu as pltpu


_GROUP = 8


def _nconv_block_kernel(a_ref, x_ref, o_ref):
    bB, _, L = x_ref.shape
    at = jnp.transpose(a_ref[...], (1, 0)).astype(jnp.bfloat16)
    g = _GROUP if bB % _GROUP == 0 else 1
    for i in range(0, bB, g):
        xg = jnp.concatenate(
            [x_ref[i + k].astype(jnp.bfloat16) for k in range(g)], axis=1)
        y = jax.lax.dot_general(
            at, xg,
            dimension_numbers=(((1,), (0,)), ((), ())),
            preferred_element_type=jnp.float32,
        )
        for k in range(g):
            o_ref[i + k] = y[:, k * L:(k + 1) * L]


def _pick_bb(bc):
    for cand in (64, 32, 16, 8, 4, 2):
        if bc % cand == 0:
            return cand
    return 1


def kernel(x, A):
    N, C, V, L = x.shape
    Va, W = A.shape
    assert Va == V
    Bc = N * C
    xb = x.reshape(Bc, V, L)
    bB = _pick_bb(Bc)
    grid = (Bc // bB,)

    itemsize = jnp.dtype(x.dtype).itemsize
    needed = 2 * bB * (V + W) * L * itemsize + 2 * V * W * itemsize
    cost = pl.CostEstimate(
        flops=2 * V * W * Bc * L,
        transcendentals=0,
        bytes_accessed=(V + W) * Bc * L * itemsize + V * W * itemsize,
    )

    out = pl.pallas_call(
        _nconv_block_kernel,
        out_shape=jax.ShapeDtypeStruct((Bc, W, L), x.dtype),
        grid=grid,
        in_specs=[
            pl.BlockSpec((V, W), lambda i: (0, 0)),
            pl.BlockSpec((bB, V, L), lambda i: (i, 0, 0)),
        ],
        out_specs=pl.BlockSpec((bB, W, L), lambda i: (i, 0, 0)),
        compiler_params=pltpu.CompilerParams(
            dimension_semantics=("parallel",),
            vmem_limit_bytes=int(needed + (6 << 20)),
        ),
        cost_estimate=cost,
    )(A, xb)
    return out.reshape(N, C, W, L)
